# baseline (speedup 1.0000x reference)
import jax
import jax.numpy as jnp
from jax import lax
from jax.experimental import pallas as pl
from jax.experimental.pallas import tpu as pltpu
from jax.experimental.pallas import tpu_sc as plsc

_ROWS = 8192
_COLS = 4096
_N = _ROWS * _COLS

_SC_ROWS = 4096
_TC_ROWS = _ROWS - _SC_ROWS
_BLOCK_ROWS = 512

_NW = 32
_WROWS = _SC_ROWS // _NW
_HALF = _COLS // 2
_NCH = (_WROWS // 8) * 2
_IT = 2 * (_HALF // 16)


def _tc_kernel(yp_ref, yt_ref, m_ref, sum_ref, cnt_ref, acc_s, acc_c):
    i = pl.program_id(0)

    @pl.when(i == 0)
    def _init():
        acc_s[...] = jnp.zeros_like(acc_s)
        acc_c[...] = jnp.zeros_like(acc_c)

    d = yp_ref[...] - yt_ref[...]
    c = m_ref[...].astype(jnp.float32)
    sq = d * d * c
    ps = sq[0:8]
    pc = c[0:8]
    for k in range(1, _BLOCK_ROWS // 8):
        ps = ps + sq[8 * k:8 * k + 8]
        pc = pc + c[8 * k:8 * k + 8]
    acc_s[...] += ps
    acc_c[...] += pc

    @pl.when(i == pl.num_programs(0) - 1)
    def _fini():
        sum_ref[0, 0] = jnp.sum(acc_s[...])
        cnt_ref[0, 0] = jnp.sum(acc_c[...])


def _tc_partials(yp, yt, m):
    grid = (_TC_ROWS // _BLOCK_ROWS,)
    in_spec = pl.BlockSpec((_BLOCK_ROWS, _COLS), lambda i: (i, 0))
    out_spec = pl.BlockSpec(memory_space=pltpu.SMEM)
    return pl.pallas_call(
        _tc_kernel,
        grid=grid,
        in_specs=[in_spec, in_spec, in_spec],
        out_specs=[out_spec, out_spec],
        out_shape=[
            jax.ShapeDtypeStruct((1, 1), jnp.float32),
            jax.ShapeDtypeStruct((1, 1), jnp.float32),
        ],
        scratch_shapes=[
            pltpu.VMEM((8, _COLS), jnp.float32),
            pltpu.VMEM((8, _COLS), jnp.float32),
        ],
    )(yp, yt, m)


def _sc_masked_mse(yp_hbm, yt_hbm, m_hbm, out_s_hbm, out_c_hbm,
                   ypb, ytb, mb, accv, cntv,
                   s_yp0, s_yp1, s_yt0, s_yt1, s_m0, s_m1):
    wid = lax.axis_index("s") * 2 + lax.axis_index("c")
    row0 = _TC_ROWS + wid * _WROWS

    fsems = ((s_yp0, s_yt0), (s_yp1, s_yt1))
    msems = (s_m0, s_m1)


    def fstart(c, b):
        band = 4 * (c >> 3) + (c & 3)
        half = (c >> 2) & 1
        r = pl.multiple_of(row0 + 8 * band, 8)
        co = pl.multiple_of(half * _HALF, _HALF)
        pltpu.async_copy(yp_hbm.at[pl.ds(r, 8), pl.ds(co, _HALF)],
                         ypb.at[b], fsems[b][0])
        pltpu.async_copy(yt_hbm.at[pl.ds(r, 8), pl.ds(co, _HALF)],
                         ytb.at[b], fsems[b][1])

    def fwait(c, b):
        band = 4 * (c >> 3) + (c & 3)
        half = (c >> 2) & 1
        r = pl.multiple_of(row0 + 8 * band, 8)
        co = pl.multiple_of(half * _HALF, _HALF)
        pltpu.make_async_copy(yp_hbm.at[pl.ds(r, 8), pl.ds(co, _HALF)],
                              ypb.at[b], fsems[b][0]).wait()
        pltpu.make_async_copy(yt_hbm.at[pl.ds(r, 8), pl.ds(co, _HALF)],
                              ytb.at[b], fsems[b][1]).wait()

    def mstart(mi):
        ms = mi & 1
        r = pl.multiple_of(wid * _WROWS + 32 * (mi >> 1), 32)
        co = pl.multiple_of((mi & 1) * _HALF, _HALF)
        pltpu.async_copy(m_hbm.at[pl.ds(r, 32), pl.ds(co, _HALF)],
                         mb.at[ms], msems[ms])

    def mwait(mi):
        ms = mi & 1
        r = pl.multiple_of(wid * _WROWS + 32 * (mi >> 1), 32)
        co = pl.multiple_of((mi & 1) * _HALF, _HALF)
        pltpu.make_async_copy(m_hbm.at[pl.ds(r, 32), pl.ds(co, _HALF)],
                              mb.at[ms], msems[ms]).wait()

    fstart(0, 0)
    fstart(1, 1)
    mstart(0)
    mstart(1)

    mbi = mb.bitcast(jnp.int32)

    def chunk_compute(c, b, accs):
        ms = (c >> 2) & 1
        mrow0 = (c & 3) * 2

        def make_body(rr):
            def body(g, carry):
                a0, a1, a2, a3, cp = carry
                co = pl.multiple_of(g * 16, 16)
                m32 = mbi[ms, mrow0 + rr, pl.ds(co, 16)]

                def term(j, a):
                    r = 4 * rr + j
                    d = ypb[b, r, pl.ds(co, 16)] - ytb[b, r, pl.ds(co, 16)]
                    if j == 0:
                        bit = m32 & 1
                    elif j == 3:
                        bit = m32 >> 24
                    else:
                        bit = (m32 >> (8 * j)) & 1
                    return a + d * d * bit.astype(jnp.float32)

                return (term(0, a0), term(1, a1), term(2, a2), term(3, a3),
                        cp + m32)
            return body

        a0, a1, a2, a3, cf = accs
        zi = jnp.zeros((16,), jnp.int32)
        for rr in (0, 1):
            a0, a1, a2, a3, cp = lax.fori_loop(
                0, _IT // 2, make_body(rr), (a0, a1, a2, a3, zi), unroll=4)
            cf = (cf + (cp & 0xFF).astype(jnp.float32)
                  + ((cp >> 8) & 0xFF).astype(jnp.float32)
                  + ((cp >> 16) & 0xFF).astype(jnp.float32)
                  + (cp >> 24).astype(jnp.float32))
        return (a0, a1, a2, a3, cf)

    z = jnp.zeros((16,), jnp.float32)
    accs = (z, z, z, z, z)

    for c in range(_NCH):
        b = c & 1
        fwait(c, b)
        if c % 4 == 0:
            mwait(c >> 2)
        if c + 2 < _NCH:
            fstart(c + 2, b)
        if (c + 2) % 4 == 0 and 8 <= c + 2 < _NCH:
            mstart((c + 2) >> 2)
        accs = chunk_compute(c, b, accs)
    a0, a1, a2, a3, cf = accs
    accv[...] = (a0 + a1) + (a2 + a3)
    cntv[...] = cf
    pltpu.sync_copy(accv, out_s_hbm.at[wid])
    pltpu.sync_copy(cntv, out_c_hbm.at[wid])


def _sc_partials(yp2, yt2, m2):
    mesh = plsc.VectorSubcoreMesh(core_axis_name="c", subcore_axis_name="s")
    f = pl.kernel(
        _sc_masked_mse,
        mesh=mesh,
        out_type=[
            jax.ShapeDtypeStruct((_NW, 16), jnp.float32),
            jax.ShapeDtypeStruct((_NW, 16), jnp.float32),
        ],
        scratch_types=[
            pltpu.VMEM((2, 8, _HALF), jnp.float32),
            pltpu.VMEM((2, 8, _HALF), jnp.float32),
            pltpu.VMEM((2, 32, _HALF), jnp.uint8),
            pltpu.VMEM((16,), jnp.float32),
            pltpu.VMEM((16,), jnp.float32),
            pltpu.SemaphoreType.DMA,
            pltpu.SemaphoreType.DMA,
            pltpu.SemaphoreType.DMA,
            pltpu.SemaphoreType.DMA,
            pltpu.SemaphoreType.DMA,
            pltpu.SemaphoreType.DMA,
        ],
        compiler_params=pltpu.CompilerParams(use_tc_tiling_on_sc=True),
    )
    return f(yp2, yt2, m2)


def kernel(y_pred, y_true, mask):
    yp = y_pred.reshape(_ROWS, _COLS)
    yt = y_true.reshape(_ROWS, _COLS)
    m = mask.reshape(_ROWS, _COLS)
    m8_tc = m[:_TC_ROWS].view(jnp.uint8)
    m8_sc = m[_TC_ROWS:].view(jnp.uint8)

    ss, sc = _sc_partials(yp, yt, m8_sc)
    ts, tc = _tc_partials(yp, yt, m8_tc)

    return (ts[0, 0] + jnp.sum(ss)) / (tc[0, 0] + jnp.sum(sc))


# speedup vs baseline: 1.0099x; 1.0099x over previous
import jax
import jax.numpy as jnp
from jax import lax
from jax.experimental import pallas as pl
from jax.experimental.pallas import tpu as pltpu
from jax.experimental.pallas import tpu_sc as plsc

_ROWS = 8192
_COLS = 4096
_N = _ROWS * _COLS

_SC_ROWS = 4096
_TC_ROWS = _ROWS - _SC_ROWS
_BLOCK_ROWS = 512

_NW = 32
_WROWS = _SC_ROWS // _NW
_HALF = _COLS // 2
_NCH = (_WROWS // 8) * 2
_IT = 2 * (_HALF // 16)


def _tc_kernel(yp_ref, yt_ref, m_ref, sum_ref, cnt_ref, acc_s, acc_c):
    i = pl.program_id(0)

    @pl.when(i == 0)
    def _init():
        acc_s[...] = jnp.zeros_like(acc_s)
        acc_c[...] = jnp.zeros_like(acc_c)

    d = yp_ref[...] - yt_ref[...]
    c = m_ref[...].astype(jnp.float32)
    sq = d * d * c
    ps = sq[0:8]
    pc = c[0:8]
    for k in range(1, _BLOCK_ROWS // 8):
        ps = ps + sq[8 * k:8 * k + 8]
        pc = pc + c[8 * k:8 * k + 8]
    acc_s[...] += ps
    acc_c[...] += pc

    @pl.when(i == pl.num_programs(0) - 1)
    def _fini():
        sum_ref[0, 0] = jnp.sum(acc_s[...])
        cnt_ref[0, 0] = jnp.sum(acc_c[...])


def _tc_partials(yp, yt, m):
    grid = (_TC_ROWS // _BLOCK_ROWS,)
    in_spec = pl.BlockSpec((_BLOCK_ROWS, _COLS), lambda i: (i, 0))
    out_spec = pl.BlockSpec(memory_space=pltpu.SMEM)
    return pl.pallas_call(
        _tc_kernel,
        grid=grid,
        in_specs=[in_spec, in_spec, in_spec],
        out_specs=[out_spec, out_spec],
        out_shape=[
            jax.ShapeDtypeStruct((1, 1), jnp.float32),
            jax.ShapeDtypeStruct((1, 1), jnp.float32),
        ],
        scratch_shapes=[
            pltpu.VMEM((8, _COLS), jnp.float32),
            pltpu.VMEM((8, _COLS), jnp.float32),
        ],
    )(yp, yt, m)


def _sc_masked_mse(yp_hbm, yt_hbm, m_hbm, out_s_hbm, out_c_hbm,
                   ypb, ytb, mb, accv, cntv,
                   s_yp0, s_yp1, s_yt0, s_yt1, s_m0, s_m1):
    wid = lax.axis_index("s") * 2 + lax.axis_index("c")
    row0 = _TC_ROWS + wid * _WROWS

    fsems = ((s_yp0, s_yt0), (s_yp1, s_yt1))
    msems = (s_m0, s_m1)


    def fstart(c, b):
        band = 4 * (c >> 3) + (c & 3)
        half = (c >> 2) & 1
        r = pl.multiple_of(row0 + 8 * band, 8)
        co = pl.multiple_of(half * _HALF, _HALF)
        pltpu.async_copy(yp_hbm.at[pl.ds(r, 8), pl.ds(co, _HALF)],
                         ypb.at[b], fsems[b][0])
        pltpu.async_copy(yt_hbm.at[pl.ds(r, 8), pl.ds(co, _HALF)],
                         ytb.at[b], fsems[b][1])

    def fwait(c, b):
        band = 4 * (c >> 3) + (c & 3)
        half = (c >> 2) & 1
        r = pl.multiple_of(row0 + 8 * band, 8)
        co = pl.multiple_of(half * _HALF, _HALF)
        pltpu.make_async_copy(yp_hbm.at[pl.ds(r, 8), pl.ds(co, _HALF)],
                              ypb.at[b], fsems[b][0]).wait()
        pltpu.make_async_copy(yt_hbm.at[pl.ds(r, 8), pl.ds(co, _HALF)],
                              ytb.at[b], fsems[b][1]).wait()

    def mstart(mi):
        ms = mi & 1
        r = pl.multiple_of(wid * _WROWS + 32 * (mi >> 1), 32)
        co = pl.multiple_of((mi & 1) * _HALF, _HALF)
        pltpu.async_copy(m_hbm.at[pl.ds(r, 32), pl.ds(co, _HALF)],
                         mb.at[ms], msems[ms])

    def mwait(mi):
        ms = mi & 1
        r = pl.multiple_of(wid * _WROWS + 32 * (mi >> 1), 32)
        co = pl.multiple_of((mi & 1) * _HALF, _HALF)
        pltpu.make_async_copy(m_hbm.at[pl.ds(r, 32), pl.ds(co, _HALF)],
                              mb.at[ms], msems[ms]).wait()

    fstart(0, 0)
    fstart(1, 1)
    mstart(0)
    mstart(1)

    mbi = mb.bitcast(jnp.int32)

    def chunk_compute(c, b, accs):
        ms = (c >> 2) & 1
        mrow0 = (c & 3) * 2

        def body(g, carry):
            a0, a1, a2, a3, c0, c1, c2, c3 = carry
            rr = g >> 7
            co = pl.multiple_of((g & 127) * 16, 16)
            m32 = mbi[ms, mrow0 + rr, pl.ds(co, 16)]

            def term(j, a, cn):
                r = 4 * rr + j
                d = ypb[b, r, pl.ds(co, 16)] - ytb[b, r, pl.ds(co, 16)]
                mf = ((m32 >> (8 * j)) & 1).astype(jnp.float32)
                return a + d * d * mf, cn + mf

            a0, c0 = term(0, a0, c0)
            a1, c1 = term(1, a1, c1)
            a2, c2 = term(2, a2, c2)
            a3, c3 = term(3, a3, c3)
            return (a0, a1, a2, a3, c0, c1, c2, c3)

        return lax.fori_loop(0, _IT, body, accs, unroll=4)

    z = jnp.zeros((16,), jnp.float32)
    accs = (z, z, z, z, z, z, z, z)

    for c in range(_NCH):
        b = c & 1
        fwait(c, b)
        if c % 4 == 0:
            mwait(c >> 2)
        if c + 2 < _NCH:
            fstart(c + 2, b)
        if (c + 2) % 4 == 0 and 8 <= c + 2 < _NCH:
            mstart((c + 2) >> 2)
        accs = chunk_compute(c, b, accs)
    a0, a1, a2, a3, c0, c1, c2, c3 = accs
    accv[...] = (a0 + a1) + (a2 + a3)
    cntv[...] = (c0 + c1) + (c2 + c3)
    pltpu.sync_copy(accv, out_s_hbm.at[wid])
    pltpu.sync_copy(cntv, out_c_hbm.at[wid])


def _sc_partials(yp2, yt2, m2):
    mesh = plsc.VectorSubcoreMesh(core_axis_name="c", subcore_axis_name="s")
    f = pl.kernel(
        _sc_masked_mse,
        mesh=mesh,
        out_type=[
            jax.ShapeDtypeStruct((_NW, 16), jnp.float32),
            jax.ShapeDtypeStruct((_NW, 16), jnp.float32),
        ],
        scratch_types=[
            pltpu.VMEM((2, 8, _HALF), jnp.float32),
            pltpu.VMEM((2, 8, _HALF), jnp.float32),
            pltpu.VMEM((2, 32, _HALF), jnp.uint8),
            pltpu.VMEM((16,), jnp.float32),
            pltpu.VMEM((16,), jnp.float32),
            pltpu.SemaphoreType.DMA,
            pltpu.SemaphoreType.DMA,
            pltpu.SemaphoreType.DMA,
            pltpu.SemaphoreType.DMA,
            pltpu.SemaphoreType.DMA,
            pltpu.SemaphoreType.DMA,
        ],
        compiler_params=pltpu.CompilerParams(use_tc_tiling_on_sc=True),
    )
    return f(yp2, yt2, m2)


def kernel(y_pred, y_true, mask):
    yp = y_pred.reshape(_ROWS, _COLS)
    yt = y_true.reshape(_ROWS, _COLS)
    m = mask.reshape(_ROWS, _COLS)
    m8_tc = m[:_TC_ROWS].view(jnp.uint8)
    m8_sc = m[_TC_ROWS:].view(jnp.uint8)

    ss, sc = _sc_partials(yp, yt, m8_sc)
    ts, tc = _tc_partials(yp, yt, m8_tc)

    return (ts[0, 0] + jnp.sum(ss)) / (tc[0, 0] + jnp.sum(sc))
